# Initial kernel scaffold; baseline (speedup 1.0000x reference)
import jax
import jax.numpy as jnp
from jax.experimental import pallas as pl


def kernel(x, e, node_table, edge_table, pos_table):
    raise NotImplementedError("write your pallas kernel here")


# speedup vs baseline: 6.6847x; 6.6847x over previous
import functools

import jax
import jax.numpy as jnp
from jax import lax
from jax.experimental import pallas as pl
from jax.experimental.pallas import tpu as pltpu
from jax.experimental.pallas import tpu_sc as plsc

_N_NODES = 10000
_N_EDGES = 320000
_NODE_DIM = 256
_EDGE_DIM = 128
_N_POS = 128
_N_ATTR = 64

_NC, _NS = 2, 16
_NW = _NC * _NS

_XP = 10240
_BX = _XP // _NW
_CX = 160
_BE = _N_EDGES // _NW
_CE = 400


def _combine_body(pos_ref, edge_ref, out_ref):
    out_ref[:] = pos_ref[:][:, None, :] + edge_ref[:][None, :, :]


def _build_combined(pos_table, edge_table):
    out3 = pl.pallas_call(
        _combine_body,
        out_shape=jax.ShapeDtypeStruct((_N_POS, _N_ATTR, _EDGE_DIM), jnp.float32),
    )(pos_table, edge_table)
    return out3.reshape(_N_POS * _N_ATTR, _EDGE_DIM)


_mesh = plsc.VectorSubcoreMesh(core_axis_name="c", subcore_axis_name="s")


@functools.partial(
    pl.kernel,
    out_type=(
        jax.ShapeDtypeStruct((_XP, _NODE_DIM), jnp.float32),
        jax.ShapeDtypeStruct((_N_EDGES, _EDGE_DIM), jnp.float32),
    ),
    mesh=_mesh,
    scratch_types=[
        pltpu.VMEM((_CX,), jnp.int32),
        pltpu.VMEM((_CX, _NODE_DIM), jnp.float32),
        pltpu.VMEM((_CE,), jnp.int32),
        pltpu.VMEM((_CE, _EDGE_DIM), jnp.float32),
        pltpu.SemaphoreType.DMA,
    ],
)
def _sc_gather(x_hbm, e_hbm, node_hbm, comb_hbm, x_out, e_out,
               xidx_v, xrows_v, eidx_v, erows_v, sem):
    wid = lax.axis_index("s") * _NC + lax.axis_index("c")

    xbase = wid * _BX
    for i in range(_BX // _CX):
        b = pl.multiple_of(xbase + i * _CX, 8)
        pltpu.sync_copy(x_hbm.at[pl.ds(b, _CX)], xidx_v)
        pltpu.async_copy(node_hbm.at[xidx_v], xrows_v, sem).wait()
        pltpu.sync_copy(xrows_v, x_out.at[pl.ds(b, _CX)])

    ebase = wid * _BE

    def ebody(i, carry):
        b = pl.multiple_of(ebase + i * _CE, 8)
        pltpu.sync_copy(e_hbm.at[pl.ds(b, _CE)], eidx_v)
        pltpu.async_copy(comb_hbm.at[eidx_v], erows_v, sem).wait()
        pltpu.sync_copy(erows_v, e_out.at[pl.ds(b, _CE)])
        return carry

    lax.fori_loop(0, _BE // _CE, ebody, 0)


def kernel(x, e, node_table, edge_table, pos_table):
    combined = _build_combined(pos_table, edge_table)
    xp = jnp.pad(x.astype(jnp.int32), (0, _XP - _N_NODES))
    x_emb_p, e_emb = _sc_gather(xp, e.astype(jnp.int32), node_table, combined)
    return (x_emb_p[:_N_NODES], e_emb)


# speedup vs baseline: 7.4118x; 1.1088x over previous
import functools

import jax
import jax.numpy as jnp
from jax import lax
from jax.experimental import pallas as pl
from jax.experimental.pallas import tpu as pltpu
from jax.experimental.pallas import tpu_sc as plsc

_N_NODES = 10000
_N_EDGES = 320000
_NODE_DIM = 256
_EDGE_DIM = 128
_N_POS = 128
_N_ATTR = 64

_NC, _NS = 2, 16
_NW = _NC * _NS

_XP = 10240
_BX = _XP // _NW
_CX = 160
_BE = _N_EDGES // _NW
_CE = 200
_NPAIR = _BE // _CE // 2


def _combine_body(pos_ref, edge_ref, out_ref):
    out_ref[:] = pos_ref[:][:, None, :] + edge_ref[:][None, :, :]


def _build_combined(pos_table, edge_table):
    out3 = pl.pallas_call(
        _combine_body,
        out_shape=jax.ShapeDtypeStruct((_N_POS, _N_ATTR, _EDGE_DIM), jnp.float32),
    )(pos_table, edge_table)
    return out3.reshape(_N_POS * _N_ATTR, _EDGE_DIM)


_mesh = plsc.VectorSubcoreMesh(core_axis_name="c", subcore_axis_name="s")


@functools.partial(
    pl.kernel,
    out_type=(
        jax.ShapeDtypeStruct((_XP, _NODE_DIM), jnp.float32),
        jax.ShapeDtypeStruct((_N_EDGES, _EDGE_DIM), jnp.float32),
    ),
    mesh=_mesh,
    scratch_types=[
        pltpu.VMEM((_CX,), jnp.int32),
        pltpu.VMEM((_CX, _NODE_DIM), jnp.float32),
        pltpu.VMEM((_CE,), jnp.int32),
        pltpu.VMEM((_CE,), jnp.int32),
        pltpu.VMEM((_CE, _EDGE_DIM), jnp.float32),
        pltpu.VMEM((_CE, _EDGE_DIM), jnp.float32),
        pltpu.SemaphoreType.DMA,
        pltpu.SemaphoreType.DMA,
        pltpu.SemaphoreType.DMA,
        pltpu.SemaphoreType.DMA,
        pltpu.SemaphoreType.DMA,
    ],
)
def _sc_gather(x_hbm, e_hbm, node_hbm, comb_hbm, x_out, e_out,
               xidx_v, xrows_v, eidx0, eidx1, erows0, erows1,
               xsem, isem0, isem1, osem0, osem1):
    wid = lax.axis_index("s") * _NC + lax.axis_index("c")

    xbase = wid * _BX
    for i in range(_BX // _CX):
        b = pl.multiple_of(xbase + i * _CX, 8)
        pltpu.sync_copy(x_hbm.at[pl.ds(b, _CX)], xidx_v)
        pltpu.async_copy(node_hbm.at[xidx_v], xrows_v, xsem).wait()
        pltpu.sync_copy(xrows_v, x_out.at[pl.ds(b, _CX)])

    ebase = wid * _BE

    def gather_in(i, idx_v, rows_v, sem):
        b = pl.multiple_of(ebase + i * _CE, 8)
        pltpu.sync_copy(e_hbm.at[pl.ds(b, _CE)], idx_v)
        return pltpu.async_copy(comb_hbm.at[idx_v], rows_v, sem)

    def copy_out(i, rows_v, sem):
        b = pl.multiple_of(ebase + i * _CE, 8)
        pltpu.async_copy(rows_v, e_out.at[pl.ds(b, _CE)], sem)

    def wait_out(rows_v, sem):
        pltpu.make_async_copy(rows_v, e_out.at[pl.ds(0, _CE)], sem).wait()

    ga = gather_in(0, eidx0, erows0, isem0)
    gb = gather_in(1, eidx1, erows1, isem1)
    ga.wait()
    copy_out(0, erows0, osem0)
    gb.wait()
    copy_out(1, erows1, osem1)

    def pair_body(g, carry):
        i0 = 2 * g
        wait_out(erows0, osem0)
        ga = gather_in(i0, eidx0, erows0, isem0)
        wait_out(erows1, osem1)
        gb = gather_in(i0 + 1, eidx1, erows1, isem1)
        ga.wait()
        copy_out(i0, erows0, osem0)
        gb.wait()
        copy_out(i0 + 1, erows1, osem1)
        return carry

    lax.fori_loop(1, _NPAIR, pair_body, 0)
    wait_out(erows0, osem0)
    wait_out(erows1, osem1)


def kernel(x, e, node_table, edge_table, pos_table):
    combined = _build_combined(pos_table, edge_table)
    xp = jnp.pad(x.astype(jnp.int32), (0, _XP - _N_NODES))
    x_emb_p, e_emb = _sc_gather(xp, e.astype(jnp.int32), node_table, combined)
    return (x_emb_p[:_N_NODES], e_emb)


# speedup vs baseline: 8.3904x; 1.1320x over previous
import functools

import jax
import jax.numpy as jnp
from jax import lax
from jax.experimental import pallas as pl
from jax.experimental.pallas import tpu as pltpu
from jax.experimental.pallas import tpu_sc as plsc

_N_NODES = 10000
_N_EDGES = 320000
_NODE_DIM = 256
_EDGE_DIM = 128
_N_POS = 128
_N_ATTR = 64

_NC, _NS = 2, 16
_NW = _NC * _NS

_XP = 10240
_BX = _XP // _NW
_CX = 40
_BE = _N_EDGES // _NW
_CE = 200
_NPAIR = _BE // _CE // 2


def _combine_body(pos_ref, edge_ref, out_ref):
    out_ref[:] = pos_ref[:][:, None, :] + edge_ref[:][None, :, :]


def _build_combined(pos_table, edge_table):
    out3 = pl.pallas_call(
        _combine_body,
        out_shape=jax.ShapeDtypeStruct((_N_POS, _N_ATTR, _EDGE_DIM), jnp.float32),
    )(pos_table, edge_table)
    return out3.reshape(_N_POS * _N_ATTR, _EDGE_DIM)


_mesh = plsc.VectorSubcoreMesh(core_axis_name="c", subcore_axis_name="s")


@functools.partial(
    pl.kernel,
    out_type=(
        jax.ShapeDtypeStruct((_XP, _NODE_DIM), jnp.float32),
        jax.ShapeDtypeStruct((_N_EDGES, _EDGE_DIM), jnp.float32),
    ),
    mesh=_mesh,
    scratch_types=[
        pltpu.VMEM((_CX,), jnp.int32),
        pltpu.VMEM((_CX, _NODE_DIM), jnp.float32),
        pltpu.VMEM((_CE,), jnp.int32),
        pltpu.VMEM((_CE,), jnp.int32),
        pltpu.VMEM((_CE, _EDGE_DIM), jnp.float32),
        pltpu.VMEM((_CE, _EDGE_DIM), jnp.float32),
        pltpu.SemaphoreType.DMA,
        pltpu.SemaphoreType.DMA,
        pltpu.SemaphoreType.DMA,
        pltpu.SemaphoreType.DMA,
        pltpu.SemaphoreType.DMA,
        pltpu.VMEM_SHARED((_N_POS * _N_ATTR, _EDGE_DIM), jnp.float32),
    ],
)
def _sc_gather(x_hbm, e_hbm, node_hbm, comb_hbm, x_out, e_out,
               xidx_v, xrows_v, eidx0, eidx1, erows0, erows1,
               xsem, isem0, isem1, osem0, osem1, comb_sh):
    wid = lax.axis_index("s") * _NC + lax.axis_index("c")
    sid = lax.axis_index("s")

    cb = pl.multiple_of(sid * (_N_POS * _N_ATTR // _NS), 8)
    pltpu.sync_copy(comb_hbm.at[pl.ds(cb, _N_POS * _N_ATTR // _NS)],
                    comb_sh.at[pl.ds(cb, _N_POS * _N_ATTR // _NS)])
    plsc.subcore_barrier()

    xbase = wid * _BX
    for i in range(_BX // _CX):
        b = pl.multiple_of(xbase + i * _CX, 8)
        pltpu.sync_copy(x_hbm.at[pl.ds(b, _CX)], xidx_v)
        pltpu.async_copy(node_hbm.at[xidx_v], xrows_v, xsem).wait()
        pltpu.sync_copy(xrows_v, x_out.at[pl.ds(b, _CX)])

    ebase = wid * _BE

    def gather_in(i, idx_v, rows_v, sem):
        b = pl.multiple_of(ebase + i * _CE, 8)
        pltpu.sync_copy(e_hbm.at[pl.ds(b, _CE)], idx_v)
        return pltpu.async_copy(comb_sh.at[idx_v], rows_v, sem)

    def copy_out(i, rows_v, sem):
        b = pl.multiple_of(ebase + i * _CE, 8)
        pltpu.async_copy(rows_v, e_out.at[pl.ds(b, _CE)], sem)

    def wait_out(rows_v, sem):
        pltpu.make_async_copy(rows_v, e_out.at[pl.ds(0, _CE)], sem).wait()

    ga = gather_in(0, eidx0, erows0, isem0)
    gb = gather_in(1, eidx1, erows1, isem1)
    ga.wait()
    copy_out(0, erows0, osem0)
    gb.wait()
    copy_out(1, erows1, osem1)

    def pair_body(g, carry):
        i0 = 2 * g
        wait_out(erows0, osem0)
        ga = gather_in(i0, eidx0, erows0, isem0)
        wait_out(erows1, osem1)
        gb = gather_in(i0 + 1, eidx1, erows1, isem1)
        ga.wait()
        copy_out(i0, erows0, osem0)
        gb.wait()
        copy_out(i0 + 1, erows1, osem1)
        return carry

    lax.fori_loop(1, _NPAIR, pair_body, 0)
    wait_out(erows0, osem0)
    wait_out(erows1, osem1)


def kernel(x, e, node_table, edge_table, pos_table):
    combined = _build_combined(pos_table, edge_table)
    xp = jnp.pad(x.astype(jnp.int32), (0, _XP - _N_NODES))
    x_emb_p, e_emb = _sc_gather(xp, e.astype(jnp.int32), node_table, combined)
    return (x_emb_p[:_N_NODES], e_emb)


# speedup vs baseline: 11.1322x; 1.3268x over previous
import functools

import jax
import jax.numpy as jnp
from jax import lax
from jax.experimental import pallas as pl
from jax.experimental.pallas import tpu as pltpu
from jax.experimental.pallas import tpu_sc as plsc

_N_NODES = 10000
_N_EDGES = 320000
_NODE_DIM = 256
_EDGE_DIM = 128
_N_POS = 128
_N_ATTR = 64

_NC, _NS = 2, 16
_NW = _NC * _NS

_XP = 10240
_BX = _XP // _NW
_CX = 40
_BE = _N_EDGES // _NW
_CE = 200
_NPAIR = _BE // _CE // 2


def _combine_body(pos_ref, edge_ref, out_ref):
    out_ref[:] = pos_ref[:][:, None, :] + edge_ref[:][None, :, :]


def _build_combined(pos_table, edge_table):
    out3 = pl.pallas_call(
        _combine_body,
        out_shape=jax.ShapeDtypeStruct((_N_POS, _N_ATTR, _EDGE_DIM), jnp.float32),
    )(pos_table, edge_table)
    return out3.reshape(_N_POS * _N_ATTR, _EDGE_DIM)


_mesh = plsc.VectorSubcoreMesh(core_axis_name="c", subcore_axis_name="s")


@functools.partial(
    pl.kernel,
    out_type=(
        jax.ShapeDtypeStruct((_N_NODES, _NODE_DIM), jnp.float32),
        jax.ShapeDtypeStruct((_N_EDGES, _EDGE_DIM), jnp.float32),
    ),
    mesh=_mesh,
    scratch_types=[
        pltpu.VMEM((_CX,), jnp.int32),
        pltpu.VMEM((_CX, _NODE_DIM), jnp.float32),
        pltpu.VMEM((_CE,), jnp.int32),
        pltpu.VMEM((_CE,), jnp.int32),
        pltpu.VMEM((_CE, _EDGE_DIM), jnp.float32),
        pltpu.VMEM((_CE, _EDGE_DIM), jnp.float32),
        pltpu.SemaphoreType.DMA,
        pltpu.SemaphoreType.DMA,
        pltpu.SemaphoreType.DMA,
        pltpu.SemaphoreType.DMA,
        pltpu.SemaphoreType.DMA,
        pltpu.VMEM_SHARED((_N_POS * _N_ATTR, _EDGE_DIM), jnp.float32),
    ],
)
def _sc_gather(x_hbm, e_hbm, node_hbm, comb_hbm, x_out, e_out,
               xidx_v, xrows_v, eidx0, eidx1, erows0, erows1,
               xsem, isem0, isem1, osem0, osem1, comb_sh):
    wid = lax.axis_index("s") * _NC + lax.axis_index("c")
    sid = lax.axis_index("s")

    cb = pl.multiple_of(sid * (_N_POS * _N_ATTR // _NS), 8)
    stage = pltpu.async_copy(comb_hbm.at[pl.ds(cb, _N_POS * _N_ATTR // _NS)],
                             comb_sh.at[pl.ds(cb, _N_POS * _N_ATTR // _NS)],
                             isem0)

    xbase = wid * _BX
    for i in range(_BX // _CX):
        b = pl.multiple_of(xbase + i * _CX, 8)

        @pl.when(b < _N_NODES)
        def _():
            pltpu.sync_copy(x_hbm.at[pl.ds(b, _CX)], xidx_v)
            pltpu.async_copy(node_hbm.at[xidx_v], xrows_v, xsem).wait()
            pltpu.sync_copy(xrows_v, x_out.at[pl.ds(b, _CX)])

    stage.wait()
    plsc.subcore_barrier()

    ebase = wid * _BE

    def gather_in(i, idx_v, rows_v, sem):
        b = pl.multiple_of(ebase + i * _CE, 8)
        pltpu.sync_copy(e_hbm.at[pl.ds(b, _CE)], idx_v)
        return pltpu.async_copy(comb_sh.at[idx_v], rows_v, sem)

    def copy_out(i, rows_v, sem):
        b = pl.multiple_of(ebase + i * _CE, 8)
        pltpu.async_copy(rows_v, e_out.at[pl.ds(b, _CE)], sem)

    def wait_out(rows_v, sem):
        pltpu.make_async_copy(rows_v, e_out.at[pl.ds(0, _CE)], sem).wait()

    ga = gather_in(0, eidx0, erows0, isem0)
    gb = gather_in(1, eidx1, erows1, isem1)
    ga.wait()
    copy_out(0, erows0, osem0)
    gb.wait()
    copy_out(1, erows1, osem1)

    def pair_body(g, carry):
        i0 = 2 * g
        wait_out(erows0, osem0)
        ga = gather_in(i0, eidx0, erows0, isem0)
        wait_out(erows1, osem1)
        gb = gather_in(i0 + 1, eidx1, erows1, isem1)
        ga.wait()
        copy_out(i0, erows0, osem0)
        gb.wait()
        copy_out(i0 + 1, erows1, osem1)
        return carry

    lax.fori_loop(1, _NPAIR, pair_body, 0)
    wait_out(erows0, osem0)
    wait_out(erows1, osem1)


def kernel(x, e, node_table, edge_table, pos_table):
    combined = _build_combined(pos_table, edge_table)
    x_emb, e_emb = _sc_gather(x.astype(jnp.int32), e.astype(jnp.int32),
                              node_table, combined)
    return (x_emb, e_emb)
